# baseline (speedup 1.0000x reference)
import functools

import jax
import jax.numpy as jnp
from jax.experimental import pallas as pl
from jax.experimental.pallas import tpu as pltpu

_CONTEXT_WINDOW = 4096


def _loopback_kernel(tok_ref, idea_ref, emb_ref, out_ref, carry_ref, *, nb):
    j = pl.program_id(1)
    r = idea_ref.shape[1]
    out_ref[0, 0:r - 1, :] = idea_ref[0, 1:r, :]

    @pl.when(j == 0)
    def _():
        sub = tok_ref[0] % 8
        vals = emb_ref[...]
        rows = jax.lax.broadcasted_iota(jnp.int32, vals.shape, 0)
        row = jnp.sum(jnp.where(rows == sub, vals, 0.0), axis=0, keepdims=True)
        out_ref[0, r - 1:r, :] = row

    @pl.when(j != 0)
    def _():
        out_ref[0, r - 1:r, :] = carry_ref[...]

    carry_ref[...] = idea_ref[0, 0:1, :]


def kernel(idea, token, emb):
    b, l, d = idea.shape
    lout = min(_CONTEXT_WINDOW, l + 1)
    if lout == l + 1:
        idea = jnp.concatenate([jnp.zeros((b, 1, d), idea.dtype), idea], axis=1)
        l = lout
    r = 512 if l % 512 == 0 else l
    nb = l // r
    tok = jnp.asarray(token, jnp.int32).reshape(1)
    grid_spec = pltpu.PrefetchScalarGridSpec(
        num_scalar_prefetch=1,
        grid=(b, nb),
        in_specs=[
            pl.BlockSpec((1, r, d), lambda bb, j, tok: (bb, nb - 1 - j, 0)),
            pl.BlockSpec((8, d), lambda bb, j, tok: (tok[0] // 8, 0)),
        ],
        out_specs=pl.BlockSpec((1, r, d), lambda bb, j, tok: (bb, nb - 1 - j, 0)),
        scratch_shapes=[pltpu.VMEM((1, d), idea.dtype)],
    )
    out = pl.pallas_call(
        functools.partial(_loopback_kernel, nb=nb),
        grid_spec=grid_spec,
        out_shape=jax.ShapeDtypeStruct((b, l, d), idea.dtype),
    )(tok, idea, emb)
    return out


# speedup vs baseline: 1.0168x; 1.0168x over previous
import functools

import jax
import jax.numpy as jnp
from jax.experimental import pallas as pl
from jax.experimental.pallas import tpu as pltpu

_CONTEXT_WINDOW = 4096


def _loopback_kernel(tok_ref, idea_ref, emb_ref, out_ref, carry_ref, *, nb):
    j = pl.program_id(1)
    r = idea_ref.shape[1]
    out_ref[0, 0:r - 1, :] = idea_ref[0, 1:r, :]

    @pl.when(j == 0)
    def _():
        sub = tok_ref[0] % 8
        vals = emb_ref[...]
        rows = jax.lax.broadcasted_iota(jnp.int32, vals.shape, 0)
        row = jnp.sum(jnp.where(rows == sub, vals, 0.0), axis=0, keepdims=True)
        out_ref[0, r - 1:r, :] = row

    @pl.when(j != 0)
    def _():
        out_ref[0, r - 1:r, :] = carry_ref[...]

    carry_ref[...] = idea_ref[0, 0:1, :]


def kernel(idea, token, emb):
    b, l, d = idea.shape
    lout = min(_CONTEXT_WINDOW, l + 1)
    if lout == l + 1:
        idea = jnp.concatenate([jnp.zeros((b, 1, d), idea.dtype), idea], axis=1)
        l = lout
    r = 1024 if l % 1024 == 0 else l
    nb = l // r
    tok = jnp.asarray(token, jnp.int32).reshape(1)
    grid_spec = pltpu.PrefetchScalarGridSpec(
        num_scalar_prefetch=1,
        grid=(b, nb),
        in_specs=[
            pl.BlockSpec((1, r, d), lambda bb, j, tok: (bb, nb - 1 - j, 0)),
            pl.BlockSpec((8, d), lambda bb, j, tok: (tok[0] // 8, 0)),
        ],
        out_specs=pl.BlockSpec((1, r, d), lambda bb, j, tok: (bb, nb - 1 - j, 0)),
        scratch_shapes=[pltpu.VMEM((1, d), idea.dtype)],
    )
    out = pl.pallas_call(
        functools.partial(_loopback_kernel, nb=nb),
        grid_spec=grid_spec,
        out_shape=jax.ShapeDtypeStruct((b, l, d), idea.dtype),
        compiler_params=pltpu.CompilerParams(
            dimension_semantics=("parallel", "arbitrary"),
            vmem_limit_bytes=100 * 1024 * 1024,
        ),
    )(tok, idea, emb)
    return out
